# Initial kernel scaffold; baseline (speedup 1.0000x reference)
import jax
import jax.numpy as jnp
from jax.experimental import pallas as pl


def kernel(feat, edge_index, W_src, W_dst, b_dst, W_res, b_res):
    raise NotImplementedError("write your pallas kernel here")


# speedup vs baseline: 3.9614x; 3.9614x over previous
import functools

import jax
import jax.numpy as jnp
from jax import lax
from jax.experimental import pallas as pl
from jax.experimental.pallas import tpu as pltpu
from jax.experimental.pallas import tpu_sc as plsc

N = 10000
E = 320000
D = 128
NW = 32
CHUNK = 128
CPW = 80
EPAD = NW * CPW * CHUNK
AGG_ROWS = 10240
ROWS_PER_TILE = AGG_ROWS // 16


def _tc1_body(feat_ref, ws_ref, wd_ref, bd_ref, sf_ref, df_ref):
    f = feat_ref[...]
    sf_ref[...] = jnp.dot(f, ws_ref[...], preferred_element_type=jnp.float32)
    df_ref[...] = (
        jnp.dot(f, wd_ref[...], preferred_element_type=jnp.float32) + bd_ref[...]
    )


def _tc1(feat, ws_t, wd_t, bd):
    grid = (10,)
    blk = 1000
    return pl.pallas_call(
        _tc1_body,
        grid=grid,
        in_specs=[
            pl.BlockSpec((blk, D), lambda i: (i, 0)),
            pl.BlockSpec((D, D), lambda i: (0, 0)),
            pl.BlockSpec((D, D), lambda i: (0, 0)),
            pl.BlockSpec((1, D), lambda i: (0, 0)),
        ],
        out_specs=[
            pl.BlockSpec((blk, D), lambda i: (i, 0)),
            pl.BlockSpec((blk, D), lambda i: (i, 0)),
        ],
        out_shape=[
            jax.ShapeDtypeStruct((N, D), jnp.float32),
            jax.ShapeDtypeStruct((N, D), jnp.float32),
        ],
    )(feat, ws_t, wd_t, bd)


def _tc2_body(aggp_ref, wr_ref, br_ref, ws_ref, wd_ref, bd_ref, s2_ref, d2_ref):
    a = aggp_ref[0] + aggp_ref[1]
    out = jnp.dot(a, wr_ref[...], preferred_element_type=jnp.float32) + br_ref[...]
    s2_ref[...] = jnp.dot(out, ws_ref[...], preferred_element_type=jnp.float32)
    d2_ref[...] = (
        jnp.dot(out, wd_ref[...], preferred_element_type=jnp.float32) + bd_ref[...]
    )


def _tc2(aggp, wr_t, br, ws_t, wd_t, bd):
    grid = (10,)
    blk = 1000
    return pl.pallas_call(
        _tc2_body,
        grid=grid,
        in_specs=[
            pl.BlockSpec((2, blk, D), lambda i: (0, i, 0)),
            pl.BlockSpec((D, D), lambda i: (0, 0)),
            pl.BlockSpec((1, D), lambda i: (0, 0)),
            pl.BlockSpec((D, D), lambda i: (0, 0)),
            pl.BlockSpec((D, D), lambda i: (0, 0)),
            pl.BlockSpec((1, D), lambda i: (0, 0)),
        ],
        out_specs=[
            pl.BlockSpec((blk, D), lambda i: (i, 0)),
            pl.BlockSpec((blk, D), lambda i: (i, 0)),
        ],
        out_shape=[
            jax.ShapeDtypeStruct((N, D), jnp.float32),
            jax.ShapeDtypeStruct((N, D), jnp.float32),
        ],
    )(aggp, wr_t, br, ws_t, wd_t, bd)


def _sc1_body(src_hbm, dst_hbm, sf_hbm, df_hbm, agg_hbm,
              sidx_v, didx_v, bufS, bufD, agg_sh, semS, semD):
    c = lax.axis_index("c")
    s = lax.axis_index("s")
    wid = c * 16 + s

    def _zrow(r, carry):
        for k in range(D // 16):
            bufS[r, pl.ds(k * 16, 16)] = jnp.zeros((16,), jnp.float32)
        return carry

    lax.fori_loop(0, CHUNK, _zrow, 0)
    for j in range(ROWS_PER_TILE // CHUNK):
        pltpu.sync_copy(bufS, agg_sh.at[pl.ds(s * ROWS_PER_TILE + j * CHUNK, CHUNK), :])
    plsc.subcore_barrier()

    base = wid * CPW
    half = CPW // 2

    def _chunk(j, carry):
        cpS = pltpu.async_copy(sf_hbm.at[sidx_v.at[j]], bufS, semS)
        cpD = pltpu.async_copy(df_hbm.at[didx_v.at[j]], bufD, semD)
        cpS.wait()
        cpD.wait()
        pltpu.sync_copy(bufS, agg_sh.at[didx_v.at[j]], add=True)
        pltpu.sync_copy(bufD, agg_sh.at[didx_v.at[j]], add=True)
        return carry

    for h in range(2):
        pltpu.sync_copy(src_hbm.at[pl.ds(base + h * half, half), :], sidx_v)
        pltpu.sync_copy(dst_hbm.at[pl.ds(base + h * half, half), :], didx_v)
        lax.fori_loop(0, half, _chunk, 0)
    plsc.subcore_barrier()

    for j in range(ROWS_PER_TILE // CHUNK):
        row = s * ROWS_PER_TILE + j * CHUNK
        pltpu.sync_copy(agg_sh.at[pl.ds(row, CHUNK), :],
                        agg_hbm.at[c, pl.ds(row, CHUNK), :])


def _sc1(src2d, dst2d, sf, df):
    mesh = plsc.VectorSubcoreMesh(core_axis_name="c", subcore_axis_name="s")
    fn = functools.partial(
        pl.kernel,
        mesh=mesh,
        out_type=jax.ShapeDtypeStruct((2, AGG_ROWS, D), jnp.float32),
        scratch_types=[
            pltpu.VMEM((CPW // 2, CHUNK), jnp.int32),
            pltpu.VMEM((CPW // 2, CHUNK), jnp.int32),
            pltpu.VMEM((CHUNK, D), jnp.float32),
            pltpu.VMEM((CHUNK, D), jnp.float32),
            pltpu.VMEM_SHARED((AGG_ROWS, D), jnp.float32),
            pltpu.SemaphoreType.DMA,
            pltpu.SemaphoreType.DMA,
        ],
    )(_sc1_body)
    return fn(src2d, dst2d, sf, df)


def _sc2_body(src_hbm, dst_hbm, s2_hbm, d2_hbm, out_hbm,
              sidx_v, didx_v, bufS, bufD, semS, semD):
    c = lax.axis_index("c")
    s = lax.axis_index("s")
    wid = c * 16 + s
    base = wid * CPW
    pltpu.sync_copy(src_hbm.at[pl.ds(base, CPW), :], sidx_v)
    pltpu.sync_copy(dst_hbm.at[pl.ds(base, CPW), :], didx_v)

    nchunks = jnp.where(wid == NW - 1, (E - (NW - 1) * CPW * CHUNK) // CHUNK, CPW)

    def _chunk(j, carry):
        cpS = pltpu.async_copy(s2_hbm.at[sidx_v.at[j]], bufS, semS)
        cpD = pltpu.async_copy(d2_hbm.at[didx_v.at[j]], bufD, semD)
        cpS.wait()
        cpD.wait()

        def _vrow(r, cc):
            for k in range(D // 16):
                sl = pl.ds(k * 16, 16)
                bufS[r, sl] = bufS[r, sl] + bufD[r, sl]
            return cc

        lax.fori_loop(0, CHUNK, _vrow, 0)
        pltpu.sync_copy(bufS, out_hbm.at[pl.ds(wid * CPW * CHUNK + j * CHUNK, CHUNK), :])
        return carry

    lax.fori_loop(0, nchunks, _chunk, 0)


def _sc2(src2d, dst2d, s2, d2):
    mesh = plsc.VectorSubcoreMesh(core_axis_name="c", subcore_axis_name="s")
    fn = functools.partial(
        pl.kernel,
        mesh=mesh,
        out_type=jax.ShapeDtypeStruct((E, D), jnp.float32),
        scratch_types=[
            pltpu.VMEM((CPW, CHUNK), jnp.int32),
            pltpu.VMEM((CPW, CHUNK), jnp.int32),
            pltpu.VMEM((CHUNK, D), jnp.float32),
            pltpu.VMEM((CHUNK, D), jnp.float32),
            pltpu.SemaphoreType.DMA,
            pltpu.SemaphoreType.DMA,
        ],
    )(_sc2_body)
    return fn(src2d, dst2d, s2, d2)


def kernel(feat, edge_index, W_src, W_dst, b_dst, W_res, b_res):
    src = edge_index[0]
    dst = edge_index[1]
    pad = EPAD - E
    src2d = jnp.concatenate([src, jnp.zeros((pad,), jnp.int32)]).reshape(
        EPAD // CHUNK, CHUNK)
    dst2d = jnp.concatenate([dst, jnp.full((pad,), N, jnp.int32)]).reshape(
        EPAD // CHUNK, CHUNK)

    ws_t = W_src.T
    wd_t = W_dst.T
    wr_t = W_res.T
    bd = b_dst.reshape(1, D)
    br = b_res.reshape(1, D)

    sf, df = _tc1(feat, ws_t, wd_t, bd)
    aggp = _sc1(src2d, dst2d, sf, df)
    s2, d2 = _tc2(aggp, wr_t, br, ws_t, wd_t, bd)
    return _sc2(src2d, dst2d, s2, d2)


# speedup vs baseline: 4.1298x; 1.0425x over previous
import functools

import jax
import jax.numpy as jnp
from jax import lax
from jax.experimental import pallas as pl
from jax.experimental.pallas import tpu as pltpu
from jax.experimental.pallas import tpu_sc as plsc

N = 10000
E = 320000
D = 128
NW = 32
CHUNK = 128
CPW = 80
EPAD = NW * CPW * CHUNK
AGG_ROWS = 10240
ROWS_PER_TILE = AGG_ROWS // 16


def _tc1_body(feat_ref, ws_ref, wd_ref, bd_ref, sf_ref, df_ref):
    f = feat_ref[...]
    sf_ref[...] = jnp.dot(f, ws_ref[...], preferred_element_type=jnp.float32)
    df_ref[...] = (
        jnp.dot(f, wd_ref[...], preferred_element_type=jnp.float32) + bd_ref[...]
    )


def _tc1(feat, ws_t, wd_t, bd):
    grid = (10,)
    blk = 1000
    return pl.pallas_call(
        _tc1_body,
        grid=grid,
        in_specs=[
            pl.BlockSpec((blk, D), lambda i: (i, 0)),
            pl.BlockSpec((D, D), lambda i: (0, 0)),
            pl.BlockSpec((D, D), lambda i: (0, 0)),
            pl.BlockSpec((1, D), lambda i: (0, 0)),
        ],
        out_specs=[
            pl.BlockSpec((blk, D), lambda i: (i, 0)),
            pl.BlockSpec((blk, D), lambda i: (i, 0)),
        ],
        out_shape=[
            jax.ShapeDtypeStruct((N, D), jnp.float32),
            jax.ShapeDtypeStruct((N, D), jnp.float32),
        ],
    )(feat, ws_t, wd_t, bd)


def _tc2_body(aggp_ref, wr_ref, br_ref, ws_ref, wd_ref, bd_ref, s2_ref, d2_ref):
    a = aggp_ref[0] + aggp_ref[1]
    out = jnp.dot(a, wr_ref[...], preferred_element_type=jnp.float32) + br_ref[...]
    s2_ref[...] = jnp.dot(out, ws_ref[...], preferred_element_type=jnp.float32)
    d2_ref[...] = (
        jnp.dot(out, wd_ref[...], preferred_element_type=jnp.float32) + bd_ref[...]
    )


def _tc2(aggp, wr_t, br, ws_t, wd_t, bd):
    grid = (10,)
    blk = 1000
    return pl.pallas_call(
        _tc2_body,
        grid=grid,
        in_specs=[
            pl.BlockSpec((2, blk, D), lambda i: (0, i, 0)),
            pl.BlockSpec((D, D), lambda i: (0, 0)),
            pl.BlockSpec((1, D), lambda i: (0, 0)),
            pl.BlockSpec((D, D), lambda i: (0, 0)),
            pl.BlockSpec((D, D), lambda i: (0, 0)),
            pl.BlockSpec((1, D), lambda i: (0, 0)),
        ],
        out_specs=[
            pl.BlockSpec((blk, D), lambda i: (i, 0)),
            pl.BlockSpec((blk, D), lambda i: (i, 0)),
        ],
        out_shape=[
            jax.ShapeDtypeStruct((N, D), jnp.float32),
            jax.ShapeDtypeStruct((N, D), jnp.float32),
        ],
    )(aggp, wr_t, br, ws_t, wd_t, bd)


def _sc1_body(src_hbm, dst_hbm, sf_hbm, df_hbm, agg_hbm,
              sidx_v, didx_v, bufS, bufD, agg_sh, semS, semD):
    c = lax.axis_index("c")
    s = lax.axis_index("s")
    wid = c * 16 + s

    def _zrow(r, carry):
        for k in range(D // 16):
            bufS[r, pl.ds(k * 16, 16)] = jnp.zeros((16,), jnp.float32)
        return carry

    lax.fori_loop(0, CHUNK, _zrow, 0)
    for j in range(ROWS_PER_TILE // CHUNK):
        pltpu.sync_copy(bufS, agg_sh.at[pl.ds(s * ROWS_PER_TILE + j * CHUNK, CHUNK), :])
    plsc.subcore_barrier()

    base = wid * CPW
    half = CPW // 2

    def _chunk(j, carry):
        cpS = pltpu.async_copy(sf_hbm.at[sidx_v.at[j]], bufS, semS)
        cpD = pltpu.async_copy(df_hbm.at[didx_v.at[j]], bufD, semD)
        cpS.wait()
        cpD.wait()
        pltpu.sync_copy(bufS, agg_sh.at[didx_v.at[j]], add=True)
        pltpu.sync_copy(bufD, agg_sh.at[didx_v.at[j]], add=True)
        return carry

    for h in range(2):
        pltpu.sync_copy(src_hbm.at[pl.ds(base + h * half, half), :], sidx_v)
        pltpu.sync_copy(dst_hbm.at[pl.ds(base + h * half, half), :], didx_v)
        lax.fori_loop(0, half, _chunk, 0)
    plsc.subcore_barrier()

    for j in range(ROWS_PER_TILE // CHUNK):
        row = s * ROWS_PER_TILE + j * CHUNK
        pltpu.sync_copy(agg_sh.at[pl.ds(row, CHUNK), :],
                        agg_hbm.at[c, pl.ds(row, CHUNK), :])


def _sc1(src2d, dst2d, sf, df):
    mesh = plsc.VectorSubcoreMesh(core_axis_name="c", subcore_axis_name="s")
    fn = functools.partial(
        pl.kernel,
        mesh=mesh,
        out_type=jax.ShapeDtypeStruct((2, AGG_ROWS, D), jnp.float32),
        scratch_types=[
            pltpu.VMEM((CPW // 2, CHUNK), jnp.int32),
            pltpu.VMEM((CPW // 2, CHUNK), jnp.int32),
            pltpu.VMEM((CHUNK, D), jnp.float32),
            pltpu.VMEM((CHUNK, D), jnp.float32),
            pltpu.VMEM_SHARED((AGG_ROWS, D), jnp.float32),
            pltpu.SemaphoreType.DMA,
            pltpu.SemaphoreType.DMA,
        ],
    )(_sc1_body)
    return fn(src2d, dst2d, sf, df)


def _sc2_body(src_hbm, dst_hbm, s2_hbm, d2_hbm, out_hbm,
              sidx_v, didx_v, bufS, bufD, semS, semD):
    c = lax.axis_index("c")
    s = lax.axis_index("s")
    wid = c * 16 + s
    base = wid * CPW
    pltpu.sync_copy(src_hbm.at[pl.ds(base, CPW), :], sidx_v)
    pltpu.sync_copy(dst_hbm.at[pl.ds(base, CPW), :], didx_v)

    nchunks = jnp.where(wid == NW - 1, (E - (NW - 1) * CPW * CHUNK) // CHUNK, CPW)

    def _chunk(j, carry):
        cpS = pltpu.async_copy(s2_hbm.at[sidx_v.at[j]], bufS, semS)
        cpD = pltpu.async_copy(d2_hbm.at[didx_v.at[j]], bufD, semD)
        cpS.wait()
        cpD.wait()

        def _vrow(r, cc):
            for k in range(D // 16):
                sl = pl.ds(k * 16, 16)
                bufS[r, sl] = bufS[r, sl] + bufD[r, sl]
            return cc

        lax.fori_loop(0, CHUNK, _vrow, 0)
        pltpu.sync_copy(bufS, out_hbm.at[pl.ds(wid * CPW * CHUNK + j * CHUNK, CHUNK), :])
        return carry

    lax.fori_loop(0, nchunks, _chunk, 0)


def _sc2(src2d, dst2d, s2, d2):
    mesh = plsc.VectorSubcoreMesh(core_axis_name="c", subcore_axis_name="s")
    fn = functools.partial(
        pl.kernel,
        mesh=mesh,
        out_type=jax.ShapeDtypeStruct((E, D), jnp.float32),
        scratch_types=[
            pltpu.VMEM((CPW, CHUNK), jnp.int32),
            pltpu.VMEM((CPW, CHUNK), jnp.int32),
            pltpu.VMEM((CHUNK, D), jnp.float32),
            pltpu.VMEM((CHUNK, D), jnp.float32),
            pltpu.SemaphoreType.DMA,
            pltpu.SemaphoreType.DMA,
        ],
    )(_sc2_body)
    return fn(src2d, dst2d, s2, d2)


def kernel(feat, edge_index, W_src, W_dst, b_dst, W_res, b_res):
    src = edge_index[0]
    dst = edge_index[1]
    pad = EPAD - E
    src2d = jnp.concatenate([src, jnp.zeros((pad,), jnp.int32)]).reshape(
        EPAD // CHUNK, CHUNK)
    pad_dst = N + jnp.arange(pad, dtype=jnp.int32) % (AGG_ROWS - N)
    dst2d = jnp.concatenate([dst, pad_dst]).reshape(EPAD // CHUNK, CHUNK)

    ws_t = W_src.T
    wd_t = W_dst.T
    wr_t = W_res.T
    bd = b_dst.reshape(1, D)
    br = b_res.reshape(1, D)

    sf, df = _tc1(feat, ws_t, wd_t, bd)
    aggp = _sc1(src2d, dst2d, sf, df)
    s2, d2 = _tc2(aggp, wr_t, br, ws_t, wd_t, bd)
    return _sc2(src2d, dst2d, s2, d2)


# speedup vs baseline: 4.7516x; 1.1506x over previous
import functools

import jax
import jax.numpy as jnp
from jax import lax
from jax.experimental import pallas as pl
from jax.experimental.pallas import tpu as pltpu
from jax.experimental.pallas import tpu_sc as plsc

N = 10000
E = 320000
D = 128
NW = 32
EPAD = 327680
C1 = 64
CPW1 = 160
C2 = 128
CPW2 = 80
AGG_ROWS = 10240
ROWS_PER_TILE = AGG_ROWS // 16


def _tc1_body(feat_ref, ws_ref, wd_ref, bd_ref, sf_ref, df_ref):
    f = feat_ref[...]
    sf_ref[...] = jnp.dot(f, ws_ref[...], preferred_element_type=jnp.float32)
    df_ref[...] = (
        jnp.dot(f, wd_ref[...], preferred_element_type=jnp.float32) + bd_ref[...]
    )


def _tc1(feat, ws_t, wd_t, bd):
    grid = (10,)
    blk = 1000
    return pl.pallas_call(
        _tc1_body,
        grid=grid,
        in_specs=[
            pl.BlockSpec((blk, D), lambda i: (i, 0)),
            pl.BlockSpec((D, D), lambda i: (0, 0)),
            pl.BlockSpec((D, D), lambda i: (0, 0)),
            pl.BlockSpec((1, D), lambda i: (0, 0)),
        ],
        out_specs=[
            pl.BlockSpec((blk, D), lambda i: (i, 0)),
            pl.BlockSpec((blk, D), lambda i: (i, 0)),
        ],
        out_shape=[
            jax.ShapeDtypeStruct((N, D), jnp.float32),
            jax.ShapeDtypeStruct((N, D), jnp.float32),
        ],
    )(feat, ws_t, wd_t, bd)


def _tc2_body(aggp_ref, wr_ref, br_ref, ws_ref, wd_ref, bd_ref, s2_ref, d2_ref):
    a = aggp_ref[0] + aggp_ref[1]
    out = jnp.dot(a, wr_ref[...], preferred_element_type=jnp.float32) + br_ref[...]
    s2_ref[...] = jnp.dot(out, ws_ref[...], preferred_element_type=jnp.float32)
    d2_ref[...] = (
        jnp.dot(out, wd_ref[...], preferred_element_type=jnp.float32) + bd_ref[...]
    )


def _tc2(aggp, wr_t, br, ws_t, wd_t, bd):
    grid = (10,)
    blk = 1000
    return pl.pallas_call(
        _tc2_body,
        grid=grid,
        in_specs=[
            pl.BlockSpec((2, blk, D), lambda i: (0, i, 0)),
            pl.BlockSpec((D, D), lambda i: (0, 0)),
            pl.BlockSpec((1, D), lambda i: (0, 0)),
            pl.BlockSpec((D, D), lambda i: (0, 0)),
            pl.BlockSpec((D, D), lambda i: (0, 0)),
            pl.BlockSpec((1, D), lambda i: (0, 0)),
        ],
        out_specs=[
            pl.BlockSpec((blk, D), lambda i: (i, 0)),
            pl.BlockSpec((blk, D), lambda i: (i, 0)),
        ],
        out_shape=[
            jax.ShapeDtypeStruct((N, D), jnp.float32),
            jax.ShapeDtypeStruct((N, D), jnp.float32),
        ],
    )(aggp, wr_t, br, ws_t, wd_t, bd)


def _sc1_body(src_hbm, dst_hbm, sf_hbm, df_hbm, agg_hbm,
              sidx_v, didx_v, bufS0, bufD0, bufS1, bufD1,
              agg_sh, semG0, semG1):
    c = lax.axis_index("c")
    s = lax.axis_index("s")
    wid = c * 16 + s

    def _zrow(r, carry):
        for k in range(D // 16):
            bufS0[r, pl.ds(k * 16, 16)] = jnp.zeros((16,), jnp.float32)
        return carry

    lax.fori_loop(0, C1, _zrow, 0)
    for j in range(ROWS_PER_TILE // C1):
        pltpu.sync_copy(bufS0, agg_sh.at[pl.ds(s * ROWS_PER_TILE + j * C1, C1), :])
    plsc.subcore_barrier()

    base = wid * CPW1
    half = CPW1 // 4
    bufs = ((bufS0, bufD0, semG0), (bufS1, bufD1, semG1))

    def _issue(j, b):
        bS, bD, sem = bufs[b]
        pltpu.async_copy(sf_hbm.at[sidx_v.at[j]], bS, sem)
        pltpu.async_copy(df_hbm.at[didx_v.at[j]], bD, sem)

    def _slot(j, b):
        bS, bD, sem = bufs[b]
        pltpu.make_async_copy(sf_hbm.at[sidx_v.at[j]], bS, sem).wait()
        pltpu.make_async_copy(df_hbm.at[didx_v.at[j]], bD, sem).wait()
        pltpu.sync_copy(bS, agg_sh.at[didx_v.at[j]], add=True)
        pltpu.sync_copy(bD, agg_sh.at[didx_v.at[j]], add=True)
        pl.when(j + 2 < half)(lambda: _issue(j + 2, b))

    def _pair(jj, carry):
        _slot(2 * jj, 0)
        _slot(2 * jj + 1, 1)
        return carry

    for h in range(4):
        pltpu.sync_copy(src_hbm.at[pl.ds(base + h * half, half), :], sidx_v)
        pltpu.sync_copy(dst_hbm.at[pl.ds(base + h * half, half), :], didx_v)
        _issue(0, 0)
        _issue(1, 1)
        lax.fori_loop(0, half // 2, _pair, 0)
    plsc.subcore_barrier()

    for j in range(ROWS_PER_TILE // C2):
        row = s * ROWS_PER_TILE + j * C2
        pltpu.sync_copy(agg_sh.at[pl.ds(row, C2), :],
                        agg_hbm.at[c, pl.ds(row, C2), :])


def _sc1(src2d, dst2d, sf, df):
    mesh = plsc.VectorSubcoreMesh(core_axis_name="c", subcore_axis_name="s")
    fn = functools.partial(
        pl.kernel,
        mesh=mesh,
        out_type=jax.ShapeDtypeStruct((2, AGG_ROWS, D), jnp.float32),
        scratch_types=[
            pltpu.VMEM((CPW1 // 4, C1), jnp.int32),
            pltpu.VMEM((CPW1 // 4, C1), jnp.int32),
            pltpu.VMEM((C1, D), jnp.float32),
            pltpu.VMEM((C1, D), jnp.float32),
            pltpu.VMEM((C1, D), jnp.float32),
            pltpu.VMEM((C1, D), jnp.float32),
            pltpu.VMEM_SHARED((AGG_ROWS, D), jnp.float32),
            pltpu.SemaphoreType.DMA,
            pltpu.SemaphoreType.DMA,
        ],
    )(_sc1_body)
    return fn(src2d, dst2d, sf, df)


def _sc2_body(src_hbm, dst_hbm, s2_hbm, d2_hbm, out_hbm,
              sidx_v, didx_v, bufS0, bufD0, bufO0, bufS1, bufD1, bufO1,
              semG0, semG1, semW0, semW1):
    c = lax.axis_index("c")
    s = lax.axis_index("s")
    wid = c * 16 + s
    base = wid * CPW2
    pltpu.sync_copy(src_hbm.at[pl.ds(base, CPW2), :], sidx_v)
    pltpu.sync_copy(dst_hbm.at[pl.ds(base, CPW2), :], didx_v)

    nchunks = jnp.where(wid == NW - 1, (E - (NW - 1) * CPW2 * C2) // C2, CPW2)
    bufs = ((bufS0, bufD0, bufO0, semG0, semW0),
            (bufS1, bufD1, bufO1, semG1, semW1))

    def _issue(j, b):
        bS, bD, _, semG, _ = bufs[b]
        pltpu.async_copy(s2_hbm.at[sidx_v.at[j]], bS, semG)
        pltpu.async_copy(d2_hbm.at[didx_v.at[j]], bD, semG)

    def _out_ref(j):
        return out_hbm.at[pl.ds(wid * CPW2 * C2 + j * C2, C2), :]

    def _slot(j, b):
        bS, bD, bO, semG, semW = bufs[b]
        pltpu.make_async_copy(s2_hbm.at[sidx_v.at[j]], bS, semG).wait()
        pltpu.make_async_copy(d2_hbm.at[didx_v.at[j]], bD, semG).wait()
        pl.when(j >= 2)(
            lambda: pltpu.make_async_copy(bO, _out_ref(j - 2), semW).wait())

        def _vrow(r, cc):
            for k in range(D // 16):
                sl = pl.ds(k * 16, 16)
                bO[r, sl] = bS[r, sl] + bD[r, sl]
            return cc

        lax.fori_loop(0, C2, _vrow, 0)
        pltpu.async_copy(bO, _out_ref(j), semW)
        pl.when(j + 2 < nchunks)(lambda: _issue(j + 2, b))

    def _pair(jj, carry):
        _slot(2 * jj, 0)
        _slot(2 * jj + 1, 1)
        return carry

    _issue(0, 0)
    _issue(1, 1)
    lax.fori_loop(0, nchunks // 2, _pair, 0)
    pltpu.make_async_copy(bufO0, _out_ref(nchunks - 2), semW0).wait()
    pltpu.make_async_copy(bufO1, _out_ref(nchunks - 1), semW1).wait()


def _sc2(src2d, dst2d, s2, d2):
    mesh = plsc.VectorSubcoreMesh(core_axis_name="c", subcore_axis_name="s")
    fn = functools.partial(
        pl.kernel,
        mesh=mesh,
        out_type=jax.ShapeDtypeStruct((E, D), jnp.float32),
        scratch_types=[
            pltpu.VMEM((CPW2, C2), jnp.int32),
            pltpu.VMEM((CPW2, C2), jnp.int32),
            pltpu.VMEM((C2, D), jnp.float32),
            pltpu.VMEM((C2, D), jnp.float32),
            pltpu.VMEM((C2, D), jnp.float32),
            pltpu.VMEM((C2, D), jnp.float32),
            pltpu.VMEM((C2, D), jnp.float32),
            pltpu.VMEM((C2, D), jnp.float32),
            pltpu.SemaphoreType.DMA,
            pltpu.SemaphoreType.DMA,
            pltpu.SemaphoreType.DMA,
            pltpu.SemaphoreType.DMA,
        ],
    )(_sc2_body)
    return fn(src2d, dst2d, s2, d2)


def kernel(feat, edge_index, W_src, W_dst, b_dst, W_res, b_res):
    src = edge_index[0]
    dst = edge_index[1]
    pad = EPAD - E
    src_p = jnp.concatenate([src, jnp.zeros((pad,), jnp.int32)])
    pad_dst = N + jnp.arange(pad, dtype=jnp.int32) % (AGG_ROWS - N)
    dst_p = jnp.concatenate([dst, pad_dst])
    src_c1 = src_p.reshape(EPAD // C1, C1)
    dst_c1 = dst_p.reshape(EPAD // C1, C1)
    src_c2 = src_p.reshape(EPAD // C2, C2)
    dst_c2 = dst_p.reshape(EPAD // C2, C2)

    ws_t = W_src.T
    wd_t = W_dst.T
    wr_t = W_res.T
    bd = b_dst.reshape(1, D)
    br = b_res.reshape(1, D)

    sf, df = _tc1(feat, ws_t, wd_t, bd)
    aggp = _sc1(src_c1, dst_c1, sf, df)
    s2, d2 = _tc2(aggp, wr_t, br, ws_t, wd_t, bd)
    return _sc2(src_c2, dst_c2, s2, d2)


# speedup vs baseline: 4.7605x; 1.0019x over previous
import functools

import jax
import jax.numpy as jnp
from jax import lax
from jax.experimental import pallas as pl
from jax.experimental.pallas import tpu as pltpu
from jax.experimental.pallas import tpu_sc as plsc

N = 10000
E = 320000
D = 128
NW = 32
EPAD = 327680
C1 = 64
CPW1 = 160
C2 = 128
CPW2 = 80
AGG_ROWS = 10240
RPT = AGG_ROWS // 16


def _tc1_body(feat_ref, ws_ref, wd_ref, bd_ref, sf_ref, df_ref):
    f = feat_ref[...]
    sf_ref[...] = jnp.dot(f, ws_ref[...], preferred_element_type=jnp.float32)
    df_ref[...] = (
        jnp.dot(f, wd_ref[...], preferred_element_type=jnp.float32) + bd_ref[...]
    )


def _tc1(feat, ws_t, wd_t, bd):
    grid = (10,)
    blk = N // 10
    return pl.pallas_call(
        _tc1_body,
        grid=grid,
        in_specs=[
            pl.BlockSpec((blk, D), lambda i: (i, 0)),
            pl.BlockSpec((D, D), lambda i: (0, 0)),
            pl.BlockSpec((D, D), lambda i: (0, 0)),
            pl.BlockSpec((1, D), lambda i: (0, 0)),
        ],
        out_specs=[
            pl.BlockSpec((blk, D), lambda i: (i, 0)),
            pl.BlockSpec((blk, D), lambda i: (i, 0)),
        ],
        out_shape=[
            jax.ShapeDtypeStruct((N, D), jnp.float32),
            jax.ShapeDtypeStruct((N, D), jnp.float32),
        ],
    )(feat, ws_t, wd_t, bd)


def _tc2_body(aggp_ref, wr_ref, br_ref, ws_ref, wd_ref, bd_ref, s2_ref, d2_ref):
    a = aggp_ref[0] + aggp_ref[1]
    out = jnp.dot(a, wr_ref[...], preferred_element_type=jnp.float32) + br_ref[...]
    s2_ref[...] = jnp.dot(out, ws_ref[...], preferred_element_type=jnp.float32)
    d2_ref[...] = (
        jnp.dot(out, wd_ref[...], preferred_element_type=jnp.float32) + bd_ref[...]
    )


def _tc2(aggp, wr_t, br, ws_t, wd_t, bd):
    grid = (10,)
    blk = N // 10
    return pl.pallas_call(
        _tc2_body,
        grid=grid,
        in_specs=[
            pl.BlockSpec((2, blk, D), lambda i: (0, i, 0)),
            pl.BlockSpec((D, D), lambda i: (0, 0)),
            pl.BlockSpec((1, D), lambda i: (0, 0)),
            pl.BlockSpec((D, D), lambda i: (0, 0)),
            pl.BlockSpec((D, D), lambda i: (0, 0)),
            pl.BlockSpec((1, D), lambda i: (0, 0)),
        ],
        out_specs=[
            pl.BlockSpec((blk, D), lambda i: (i, 0)),
            pl.BlockSpec((blk, D), lambda i: (i, 0)),
        ],
        out_shape=[
            jax.ShapeDtypeStruct((N, D), jnp.float32),
            jax.ShapeDtypeStruct((N, D), jnp.float32),
        ],
    )(aggp, wr_t, br, ws_t, wd_t, bd)


def _sc1_body(src_hbm, dst_hbm, sf_hbm, df_hbm, agg_hbm,
              sidx_v, didx_v, bufS0, bufD0, bufS1, bufD1, agg_sh,
              semG0, semG1):
    c = lax.axis_index("c")
    s = lax.axis_index("s")
    wid = c * 16 + s

    def _zrow(r, carry):
        for k in range(D // 16):
            bufS0[r, pl.ds(k * 16, 16)] = jnp.zeros((16,), jnp.float32)
        return carry

    lax.fori_loop(0, C1, _zrow, 0)
    for j in range(RPT // C1):
        pltpu.sync_copy(bufS0, agg_sh.at[pl.ds(s * RPT + j * C1, C1), :])
    plsc.subcore_barrier()

    base = wid * CPW1
    half = CPW1 // 4
    bufs = ((bufS0, bufD0, semG0), (bufS1, bufD1, semG1))

    def _issue(j, b):
        bS, bD, semG = bufs[b]
        pltpu.async_copy(sf_hbm.at[sidx_v.at[j]], bS, semG)
        pltpu.async_copy(df_hbm.at[didx_v.at[j]], bD, semG)

    def _slot(j, b):
        bS, bD, semG = bufs[b]
        pltpu.make_async_copy(sf_hbm.at[sidx_v.at[j]], bS, semG).wait()
        pltpu.make_async_copy(df_hbm.at[didx_v.at[j]], bD, semG).wait()

        def _vrow(r, cc):
            for k in range(D // 16):
                sl = pl.ds(k * 16, 16)
                bS[r, sl] = bS[r, sl] + bD[r, sl]
            return cc

        lax.fori_loop(0, C1, _vrow, 0)
        pltpu.sync_copy(bS, agg_sh.at[didx_v.at[j]], add=True)
        pl.when(j + 2 < half)(lambda: _issue(j + 2, b))

    def _pair(jj, carry):
        _slot(2 * jj, 0)
        _slot(2 * jj + 1, 1)
        return carry

    for h in range(4):
        pltpu.sync_copy(src_hbm.at[pl.ds(base + h * half, half), :], sidx_v)
        pltpu.sync_copy(dst_hbm.at[pl.ds(base + h * half, half), :], didx_v)
        _issue(0, 0)
        _issue(1, 1)
        lax.fori_loop(0, half // 2, _pair, 0)
    plsc.subcore_barrier()

    for j in range(RPT // C2):
        row = s * RPT + j * C2
        pltpu.sync_copy(agg_sh.at[pl.ds(row, C2), :],
                        agg_hbm.at[c, pl.ds(row, C2), :])


def _sc1(src2d, dst2d, sf, df):
    mesh = plsc.VectorSubcoreMesh(core_axis_name="c", subcore_axis_name="s")
    fn = functools.partial(
        pl.kernel,
        mesh=mesh,
        out_type=jax.ShapeDtypeStruct((2, AGG_ROWS, D), jnp.float32),
        scratch_types=[
            pltpu.VMEM((CPW1 // 4, C1), jnp.int32),
            pltpu.VMEM((CPW1 // 4, C1), jnp.int32),
            pltpu.VMEM((C1, D), jnp.float32),
            pltpu.VMEM((C1, D), jnp.float32),
            pltpu.VMEM((C1, D), jnp.float32),
            pltpu.VMEM((C1, D), jnp.float32),
            pltpu.VMEM_SHARED((AGG_ROWS, D), jnp.float32),
            pltpu.SemaphoreType.DMA,
            pltpu.SemaphoreType.DMA,
        ],
    )(_sc1_body)
    return fn(src2d, dst2d, sf, df)


def _sc2_body(src_hbm, dst_hbm, s2_hbm, d2_hbm, out_hbm,
              sidx_v, didx_v, bufS0, bufD0, bufO0, bufS1, bufD1, bufO1,
              semG0, semG1, semW0, semW1):
    c = lax.axis_index("c")
    s = lax.axis_index("s")
    wid = c * 16 + s
    base = wid * CPW2
    pltpu.sync_copy(src_hbm.at[pl.ds(base, CPW2), :], sidx_v)
    pltpu.sync_copy(dst_hbm.at[pl.ds(base, CPW2), :], didx_v)

    nchunks = jnp.where(wid == NW - 1, (E - (NW - 1) * CPW2 * C2) // C2, CPW2)
    bufs = ((bufS0, bufD0, bufO0, semG0, semW0),
            (bufS1, bufD1, bufO1, semG1, semW1))

    def _issue(j, b):
        bS, bD, _, semG, _ = bufs[b]
        pltpu.async_copy(s2_hbm.at[sidx_v.at[j]], bS, semG)
        pltpu.async_copy(d2_hbm.at[didx_v.at[j]], bD, semG)

    def _out_ref(j):
        return out_hbm.at[pl.ds(wid * CPW2 * C2 + j * C2, C2), :]

    def _slot(j, b):
        bS, bD, bO, semG, semW = bufs[b]
        pltpu.make_async_copy(s2_hbm.at[sidx_v.at[j]], bS, semG).wait()
        pltpu.make_async_copy(d2_hbm.at[didx_v.at[j]], bD, semG).wait()
        pl.when(j >= 2)(
            lambda: pltpu.make_async_copy(bO, _out_ref(j - 2), semW).wait())

        def _vrow(r, cc):
            for k in range(D // 16):
                sl = pl.ds(k * 16, 16)
                bO[r, sl] = bS[r, sl] + bD[r, sl]
            return cc

        lax.fori_loop(0, C2, _vrow, 0)
        pltpu.async_copy(bO, _out_ref(j), semW)
        pl.when(j + 2 < nchunks)(lambda: _issue(j + 2, b))

    def _pair(jj, carry):
        _slot(2 * jj, 0)
        _slot(2 * jj + 1, 1)
        return carry

    _issue(0, 0)
    _issue(1, 1)
    lax.fori_loop(0, nchunks // 2, _pair, 0)
    pltpu.make_async_copy(bufO0, _out_ref(nchunks - 2), semW0).wait()
    pltpu.make_async_copy(bufO1, _out_ref(nchunks - 1), semW1).wait()


def _sc2(src2d, dst2d, s2, d2):
    mesh = plsc.VectorSubcoreMesh(core_axis_name="c", subcore_axis_name="s")
    fn = functools.partial(
        pl.kernel,
        mesh=mesh,
        out_type=jax.ShapeDtypeStruct((E, D), jnp.float32),
        scratch_types=[
            pltpu.VMEM((CPW2, C2), jnp.int32),
            pltpu.VMEM((CPW2, C2), jnp.int32),
            pltpu.VMEM((C2, D), jnp.float32),
            pltpu.VMEM((C2, D), jnp.float32),
            pltpu.VMEM((C2, D), jnp.float32),
            pltpu.VMEM((C2, D), jnp.float32),
            pltpu.VMEM((C2, D), jnp.float32),
            pltpu.VMEM((C2, D), jnp.float32),
            pltpu.SemaphoreType.DMA,
            pltpu.SemaphoreType.DMA,
            pltpu.SemaphoreType.DMA,
            pltpu.SemaphoreType.DMA,
        ],
    )(_sc2_body)
    return fn(src2d, dst2d, s2, d2)


def kernel(feat, edge_index, W_src, W_dst, b_dst, W_res, b_res):
    src = edge_index[0]
    dst = edge_index[1]
    pad = EPAD - E
    src_p = jnp.concatenate([src, jnp.zeros((pad,), jnp.int32)])
    pad_dst = N + jnp.arange(pad, dtype=jnp.int32) % (AGG_ROWS - N)
    dst_p = jnp.concatenate([dst, pad_dst])
    src_c1 = src_p.reshape(EPAD // C1, C1)
    dst_c1 = dst_p.reshape(EPAD // C1, C1)
    src_c2 = src_p.reshape(EPAD // C2, C2)
    dst_c2 = dst_p.reshape(EPAD // C2, C2)

    ws_t = W_src.T
    wd_t = W_dst.T
    wr_t = W_res.T
    bd = b_dst.reshape(1, D)
    br = b_res.reshape(1, D)

    sf, df = _tc1(feat, ws_t, wd_t, bd)
    aggp = _sc1(src_c1, dst_c1, sf, df)
    s2, d2 = _tc2(aggp, wr_t, br, ws_t, wd_t, bd)
    return _sc2(src_c2, dst_c2, s2, d2)


# speedup vs baseline: 5.1167x; 1.0748x over previous
import functools

import jax
import jax.numpy as jnp
from jax import lax
from jax.experimental import pallas as pl
from jax.experimental.pallas import tpu as pltpu
from jax.experimental.pallas import tpu_sc as plsc

N = 10000
E = 320000
D = 128
NW = 32
EPAD = 327680
C1 = 64
CPW1 = 160
C2 = 128
CPW2 = 80
AGG_ROWS = 10240
RPT = AGG_ROWS // 16


def _tc1_body(feat_ref, ws_ref, wd_ref, bd_ref, sf_ref, df_ref):
    f = feat_ref[...]
    sf_ref[...] = jnp.dot(f, ws_ref[...], preferred_element_type=jnp.float32)
    df_ref[...] = (
        jnp.dot(f, wd_ref[...], preferred_element_type=jnp.float32) + bd_ref[...]
    )


def _tc1(feat, ws_t, wd_t, bd):
    grid = (10,)
    blk = N // 10
    return pl.pallas_call(
        _tc1_body,
        grid=grid,
        in_specs=[
            pl.BlockSpec((blk, D), lambda i: (i, 0)),
            pl.BlockSpec((D, D), lambda i: (0, 0)),
            pl.BlockSpec((D, D), lambda i: (0, 0)),
            pl.BlockSpec((1, D), lambda i: (0, 0)),
        ],
        out_specs=[
            pl.BlockSpec((blk, D), lambda i: (i, 0)),
            pl.BlockSpec((blk, D), lambda i: (i, 0)),
        ],
        out_shape=[
            jax.ShapeDtypeStruct((N, D), jnp.float32),
            jax.ShapeDtypeStruct((N, D), jnp.float32),
        ],
    )(feat, ws_t, wd_t, bd)


def _tc2_body(aggp_ref, wr_ref, br_ref, ws_ref, wd_ref, bd_ref, s2_ref, d2_ref):
    a = aggp_ref[0] + aggp_ref[1]
    out = jnp.dot(a, wr_ref[...], preferred_element_type=jnp.float32) + br_ref[...]
    s2_ref[...] = jnp.dot(out, ws_ref[...], preferred_element_type=jnp.float32)
    d2_ref[...] = (
        jnp.dot(out, wd_ref[...], preferred_element_type=jnp.float32) + bd_ref[...]
    )


def _tc2(aggp, wr_t, br, ws_t, wd_t, bd):
    grid = (10,)
    blk = N // 10
    return pl.pallas_call(
        _tc2_body,
        grid=grid,
        in_specs=[
            pl.BlockSpec((2, blk, D), lambda i: (0, i, 0)),
            pl.BlockSpec((D, D), lambda i: (0, 0)),
            pl.BlockSpec((1, D), lambda i: (0, 0)),
            pl.BlockSpec((D, D), lambda i: (0, 0)),
            pl.BlockSpec((D, D), lambda i: (0, 0)),
            pl.BlockSpec((1, D), lambda i: (0, 0)),
        ],
        out_specs=[
            pl.BlockSpec((blk, D), lambda i: (i, 0)),
            pl.BlockSpec((blk, D), lambda i: (i, 0)),
        ],
        out_shape=[
            jax.ShapeDtypeStruct((N, D), jnp.float32),
            jax.ShapeDtypeStruct((N, D), jnp.float32),
        ],
    )(aggp, wr_t, br, ws_t, wd_t, bd)


STAGE1 = 32


def _sc1_body(cidx_hbm, didx_hbm, t_hbm, agg_hbm,
              cidx_v, didx_v, buf0, buf1, agg_sh, semG0, semG1):
    c = lax.axis_index("c")
    s = lax.axis_index("s")
    wid = c * 16 + s

    def _zrow(r, carry):
        for k in range(D // 16):
            buf0[r, pl.ds(k * 16, 16)] = jnp.zeros((16,), jnp.float32)
        return carry

    lax.fori_loop(0, 2 * C1, _zrow, 0)
    for j in range(RPT // (2 * C1)):
        pltpu.sync_copy(buf0, agg_sh.at[pl.ds(s * RPT + j * 2 * C1, 2 * C1), :])
    plsc.subcore_barrier()

    base = wid * CPW1
    bufs = ((buf0, semG0), (buf1, semG1))

    def _issue(j, b):
        bf, semG = bufs[b]
        pltpu.async_copy(t_hbm.at[cidx_v.at[j]], bf, semG)

    def _slot(j, b):
        bf, semG = bufs[b]
        pltpu.make_async_copy(t_hbm.at[cidx_v.at[j]], bf, semG).wait()

        def _vrow(r, cc):
            for k in range(D // 16):
                sl = pl.ds(k * 16, 16)
                bf[r, sl] = bf[r, sl] + bf[C1 + r, sl]
            return cc

        lax.fori_loop(0, C1, _vrow, 0)
        pltpu.sync_copy(bf.at[pl.ds(0, C1), :], agg_sh.at[didx_v.at[j]],
                        add=True)
        pl.when(j + 2 < STAGE1)(lambda: _issue(j + 2, b))

    def _pair(jj, carry):
        _slot(2 * jj, 0)
        _slot(2 * jj + 1, 1)
        return carry

    for h in range(CPW1 // STAGE1):
        pltpu.sync_copy(cidx_hbm.at[pl.ds(base + h * STAGE1, STAGE1), :], cidx_v)
        pltpu.sync_copy(didx_hbm.at[pl.ds(base + h * STAGE1, STAGE1), :], didx_v)
        _issue(0, 0)
        _issue(1, 1)
        lax.fori_loop(0, STAGE1 // 2, _pair, 0)
    plsc.subcore_barrier()

    for j in range(RPT // C2):
        row = s * RPT + j * C2
        pltpu.sync_copy(agg_sh.at[pl.ds(row, C2), :],
                        agg_hbm.at[c, pl.ds(row, C2), :])


def _sc1(cidx2d, didx2d, t):
    mesh = plsc.VectorSubcoreMesh(core_axis_name="c", subcore_axis_name="s")
    fn = functools.partial(
        pl.kernel,
        mesh=mesh,
        out_type=jax.ShapeDtypeStruct((2, AGG_ROWS, D), jnp.float32),
        scratch_types=[
            pltpu.VMEM((STAGE1, 2 * C1), jnp.int32),
            pltpu.VMEM((STAGE1, C1), jnp.int32),
            pltpu.VMEM((2 * C1, D), jnp.float32),
            pltpu.VMEM((2 * C1, D), jnp.float32),
            pltpu.VMEM_SHARED((AGG_ROWS, D), jnp.float32),
            pltpu.SemaphoreType.DMA,
            pltpu.SemaphoreType.DMA,
        ],
    )(_sc1_body)
    return fn(cidx2d, didx2d, t)


def _sc2_body(src_hbm, dst_hbm, s2_hbm, d2_hbm, out_hbm,
              sidx_v, didx_v, bufS0, bufD0, bufO0, bufS1, bufD1, bufO1,
              semG0, semG1, semW0, semW1):
    c = lax.axis_index("c")
    s = lax.axis_index("s")
    wid = c * 16 + s
    base = wid * CPW2
    pltpu.sync_copy(src_hbm.at[pl.ds(base, CPW2), :], sidx_v)
    pltpu.sync_copy(dst_hbm.at[pl.ds(base, CPW2), :], didx_v)

    nchunks = jnp.where(wid == NW - 1, (E - (NW - 1) * CPW2 * C2) // C2, CPW2)
    bufs = ((bufS0, bufD0, bufO0, semG0, semW0),
            (bufS1, bufD1, bufO1, semG1, semW1))

    def _issue(j, b):
        bS, bD, _, semG, _ = bufs[b]
        pltpu.async_copy(s2_hbm.at[sidx_v.at[j]], bS, semG)
        pltpu.async_copy(d2_hbm.at[didx_v.at[j]], bD, semG)

    def _out_ref(j):
        return out_hbm.at[pl.ds(wid * CPW2 * C2 + j * C2, C2), :]

    def _slot(j, b):
        bS, bD, bO, semG, semW = bufs[b]
        pltpu.make_async_copy(s2_hbm.at[sidx_v.at[j]], bS, semG).wait()
        pltpu.make_async_copy(d2_hbm.at[didx_v.at[j]], bD, semG).wait()
        pl.when(j >= 2)(
            lambda: pltpu.make_async_copy(bO, _out_ref(j - 2), semW).wait())

        def _vrow(r, cc):
            for k in range(D // 16):
                sl = pl.ds(k * 16, 16)
                bO[r, sl] = bS[r, sl] + bD[r, sl]
            return cc

        lax.fori_loop(0, C2, _vrow, 0)
        pltpu.async_copy(bO, _out_ref(j), semW)
        pl.when(j + 2 < nchunks)(lambda: _issue(j + 2, b))

    def _pair(jj, carry):
        _slot(2 * jj, 0)
        _slot(2 * jj + 1, 1)
        return carry

    _issue(0, 0)
    _issue(1, 1)
    lax.fori_loop(0, nchunks // 2, _pair, 0)
    pltpu.make_async_copy(bufO0, _out_ref(nchunks - 2), semW0).wait()
    pltpu.make_async_copy(bufO1, _out_ref(nchunks - 1), semW1).wait()


def _sc2(src2d, dst2d, s2, d2):
    mesh = plsc.VectorSubcoreMesh(core_axis_name="c", subcore_axis_name="s")
    fn = functools.partial(
        pl.kernel,
        mesh=mesh,
        out_type=jax.ShapeDtypeStruct((E, D), jnp.float32),
        scratch_types=[
            pltpu.VMEM((CPW2, C2), jnp.int32),
            pltpu.VMEM((CPW2, C2), jnp.int32),
            pltpu.VMEM((C2, D), jnp.float32),
            pltpu.VMEM((C2, D), jnp.float32),
            pltpu.VMEM((C2, D), jnp.float32),
            pltpu.VMEM((C2, D), jnp.float32),
            pltpu.VMEM((C2, D), jnp.float32),
            pltpu.VMEM((C2, D), jnp.float32),
            pltpu.SemaphoreType.DMA,
            pltpu.SemaphoreType.DMA,
            pltpu.SemaphoreType.DMA,
            pltpu.SemaphoreType.DMA,
        ],
    )(_sc2_body)
    return fn(src2d, dst2d, s2, d2)


def kernel(feat, edge_index, W_src, W_dst, b_dst, W_res, b_res):
    src = edge_index[0]
    dst = edge_index[1]
    pad = EPAD - E
    src_p = jnp.concatenate([src, jnp.zeros((pad,), jnp.int32)])
    pad_dst = N + jnp.arange(pad, dtype=jnp.int32) % (AGG_ROWS - N)
    dst_p = jnp.concatenate([dst, pad_dst])
    src_c1 = src_p.reshape(EPAD // C1, C1)
    dst_c1 = dst_p.reshape(EPAD // C1, C1)
    src_c2 = src_p.reshape(EPAD // C2, C2)
    dst_c2 = dst_p.reshape(EPAD // C2, C2)

    ws_t = W_src.T
    wd_t = W_dst.T
    wr_t = W_res.T
    bd = b_dst.reshape(1, D)
    br = b_res.reshape(1, D)

    sf, df = _tc1(feat, ws_t, wd_t, bd)
    t = jnp.concatenate([sf, df])
    dst_g = jnp.where(dst_p < N, dst_p + N, N)
    cidx_c1 = jnp.concatenate(
        [src_p.reshape(EPAD // C1, C1), dst_g.reshape(EPAD // C1, C1)], axis=1)
    aggp = _sc1(cidx_c1, dst_c1, t)
    s2, d2 = _tc2(aggp, wr_t, br, ws_t, wd_t, bd)
    return _sc2(src_c2, dst_c2, s2, d2)
